# Initial kernel scaffold; baseline (speedup 1.0000x reference)
import jax
import jax.numpy as jnp
from jax.experimental import pallas as pl


def kernel(x, edge_index, Wl1, Wr1, att1, bias1, bn_w, bn_b, Wl2, Wr2, att2, bias2):
    raise NotImplementedError("write your pallas kernel here")


# speedup vs baseline: 18.9136x; 18.9136x over previous
import functools

import jax
import jax.numpy as jnp
from jax import lax
from jax.experimental import pallas as pl
from jax.experimental.pallas import tpu as pltpu
from jax.experimental.pallas import tpu_sc as plsc

NEG = 0.2
BN_EPS = 1e-5
NC = 2
NS = 16
LANES = 16

_HI = lax.Precision.HIGHEST


def _tc_lin(x, Wl, Wr):
    n, din = x.shape
    dout = Wl.shape[1]
    bn = 1000
    grid = (n // bn,)

    def body(x_ref, wl_ref, wr_ref, xl_ref, xr_ref):
        xb = x_ref[...]
        xl_ref[...] = lax.dot(xb, wl_ref[...], precision=_HI,
                              preferred_element_type=jnp.float32)
        xr_ref[...] = lax.dot(xb, wr_ref[...], precision=_HI,
                              preferred_element_type=jnp.float32)

    return pl.pallas_call(
        body,
        grid=grid,
        in_specs=[
            pl.BlockSpec((bn, din), lambda i: (i, 0)),
            pl.BlockSpec((din, dout), lambda i: (0, 0)),
            pl.BlockSpec((din, dout), lambda i: (0, 0)),
        ],
        out_specs=[
            pl.BlockSpec((bn, dout), lambda i: (i, 0)),
            pl.BlockSpec((bn, dout), lambda i: (i, 0)),
        ],
        out_shape=[
            jax.ShapeDtypeStruct((n, dout), jnp.float32),
            jax.ShapeDtypeStruct((n, dout), jnp.float32),
        ],
    )(x, Wl, Wr)


def _tc_combine1(xl1, xr1, num0, num1, den0, den1, A1, E8, scale, shift,
                 Wl2, Wr2):
    n = xl1.shape[0]
    bn = 1000
    grid = (n // bn,)

    def body(xl_ref, xr_ref, n0_ref, n1_ref, d0_ref, d1_ref, a1_ref, e8_ref,
             sc_ref, sh_ref, wl2_ref, wr2_ref, xl2_ref, xr2_ref):
        xl = xl_ref[...]
        t = xl + xr_ref[...]
        t = jnp.maximum(t, NEG * t)
        p = jnp.exp(lax.dot(t, a1_ref[...], precision=_HI,
                            preferred_element_type=jnp.float32))
        den8 = d0_ref[...] + d1_ref[...] + p
        e8 = e8_ref[...]
        num = n0_ref[...] + n1_ref[...] + lax.dot(
            p, e8, precision=_HI, preferred_element_type=jnp.float32) * xl
        den = lax.dot(den8, e8, precision=_HI,
                      preferred_element_type=jnp.float32)
        o = num / (den + 1e-16)
        o = o * sc_ref[...] + sh_ref[...]
        h = jnp.where(o > 0, o, jnp.exp(o) - 1.0)
        xl2_ref[...] = lax.dot(h, wl2_ref[...], precision=_HI,
                               preferred_element_type=jnp.float32)
        xr2_ref[...] = lax.dot(h, wr2_ref[...], precision=_HI,
                               preferred_element_type=jnp.float32)

    full = lambda shape: pl.BlockSpec(shape, lambda i: tuple(0 for _ in shape))
    return pl.pallas_call(
        body,
        grid=grid,
        in_specs=[
            pl.BlockSpec((bn, 128), lambda i: (i, 0)),
            pl.BlockSpec((bn, 128), lambda i: (i, 0)),
            pl.BlockSpec((bn, 128), lambda i: (i, 0)),
            pl.BlockSpec((bn, 128), lambda i: (i, 0)),
            pl.BlockSpec((bn, 8), lambda i: (i, 0)),
            pl.BlockSpec((bn, 8), lambda i: (i, 0)),
            full((128, 8)),
            full((8, 128)),
            full((1, 128)),
            full((1, 128)),
            full((128, 64)),
            full((128, 64)),
        ],
        out_specs=[
            pl.BlockSpec((bn, 64), lambda i: (i, 0)),
            pl.BlockSpec((bn, 64), lambda i: (i, 0)),
        ],
        out_shape=[
            jax.ShapeDtypeStruct((n, 64), jnp.float32),
            jax.ShapeDtypeStruct((n, 64), jnp.float32),
        ],
    )(xl1, xr1, num0, num1, den0, den1, A1, E8, scale, shift, Wl2, Wr2)


def _tc_final(xl2, xr2, num0, num1, den0, den1, att2, bias2):
    n = xl2.shape[0]
    bn = 1000
    grid = (n // bn,)

    def body(xl_ref, xr_ref, n0_ref, n1_ref, d0_ref, d1_ref, a_ref, b_ref,
             out_ref):
        xl = xl_ref[...]
        t = xl + xr_ref[...]
        t = jnp.maximum(t, NEG * t)
        a = jnp.sum(t * a_ref[...], axis=1, keepdims=True)
        p = jnp.exp(a)
        den = d0_ref[...] + d1_ref[...] + p
        num = n0_ref[...] + n1_ref[...] + p * xl
        o = num / (den + 1e-16) + b_ref[...]
        m = jnp.max(o, axis=1, keepdims=True)
        lse = m + jnp.log(jnp.sum(jnp.exp(o - m), axis=1, keepdims=True))
        out_ref[...] = o - lse

    full = lambda shape: pl.BlockSpec(shape, lambda i: tuple(0 for _ in shape))
    return pl.pallas_call(
        body,
        grid=grid,
        in_specs=[
            pl.BlockSpec((bn, 64), lambda i: (i, 0)),
            pl.BlockSpec((bn, 64), lambda i: (i, 0)),
            pl.BlockSpec((bn, 64), lambda i: (i, 0)),
            pl.BlockSpec((bn, 64), lambda i: (i, 0)),
            pl.BlockSpec((bn, 1), lambda i: (i, 0)),
            pl.BlockSpec((bn, 1), lambda i: (i, 0)),
            full((1, 64)),
            full((1, 64)),
        ],
        out_specs=pl.BlockSpec((bn, 64), lambda i: (i, 0)),
        out_shape=jax.ShapeDtypeStruct((n, 64), jnp.float32),
    )(xl2, xr2, num0, num1, den0, den1, att2, bias2)


_GDN = lax.GatherDimensionNumbers(
    offset_dims=(), collapsed_slice_dims=(0,), start_index_map=(0,))


def _lane_shuffle(u, idx):
    return lax.gather(u, idx[:, None], _GDN, (1,),
                      mode=lax.GatherScatterMode.PROMISE_IN_BOUNDS)


def _vsum_bcast(u, iota):
    for k in (1, 2, 4, 8):
        u = u + _lane_shuffle(u, iota ^ k)
    return u

def _sc_edges(xl, xr, src, dst, att, heads, chunk, single_head):
    n = xl.shape[0]
    dim = heads * LANES
    e = src.shape[0]
    ew = e // (NC * NS)
    nchunks = ew // chunk
    rows = n // NS

    mesh = plsc.VectorSubcoreMesh(core_axis_name="c", subcore_axis_name="s")

    @functools.partial(
        pl.kernel,
        out_type=[
            jax.ShapeDtypeStruct((NC, n, dim), jnp.float32),
            jax.ShapeDtypeStruct((NC, n, LANES), jnp.float32),
        ],
        mesh=mesh,
        compiler_params=pltpu.CompilerParams(use_tc_tiling_on_sc=False),
        scratch_types=[
            pltpu.VMEM((chunk,), jnp.int32),
            pltpu.VMEM((chunk,), jnp.int32),
            pltpu.VMEM((chunk, dim), jnp.float32),
            pltpu.VMEM((chunk, dim), jnp.float32),
            pltpu.VMEM((chunk, LANES), jnp.float32),
            pltpu.VMEM((heads, LANES), jnp.float32),
            pltpu.VMEM_SHARED((n, dim), jnp.float32),
            pltpu.VMEM_SHARED((n, LANES), jnp.float32),
            pltpu.SemaphoreType.DMA,
            pltpu.SemaphoreType.DMA,
        ],
    )
    def k(xl_hbm, xr_hbm, src_hbm, dst_hbm, att_hbm, num_out, den_out,
          src_i, dst_i, xl_b, xr_b, den_b, att_v,
          num_sh, den_sh, sem1, sem2):
        c = lax.axis_index("c")
        s = lax.axis_index("s")
        wid = c * NS + s

        pltpu.sync_copy(att_hbm, att_v)

        zero = jnp.zeros((LANES,), jnp.float32)

        def zrow(i, _):
            for j in range(dim // LANES):
                xl_b[i, pl.ds(j * LANES, LANES)] = zero
            den_b[i, :] = zero
            return 0

        lax.fori_loop(0, chunk, zrow, 0, unroll=False)

        nf, zrem = divmod(rows, chunk)

        def zcopy(i, _):
            pltpu.sync_copy(xl_b, num_sh.at[pl.ds(s * rows + i * chunk,
                                                  chunk)])
            pltpu.sync_copy(den_b, den_sh.at[pl.ds(s * rows + i * chunk,
                                                   chunk)])
            return 0

        lax.fori_loop(0, nf, zcopy, 0, unroll=False)
        if zrem:
            pltpu.sync_copy(xl_b.at[pl.ds(0, zrem)],
                            num_sh.at[pl.ds(s * rows + nf * chunk, zrem)])
            pltpu.sync_copy(den_b.at[pl.ds(0, zrem)],
                            den_sh.at[pl.ds(s * rows + nf * chunk, zrem)])
        plsc.subcore_barrier()

        iota = lax.iota(jnp.int32, LANES)
        base_w = wid * ew

        def chunk_body(ci, _):
            base = base_w + ci * chunk
            pltpu.sync_copy(src_hbm.at[pl.ds(base, chunk)], src_i)
            pltpu.sync_copy(dst_hbm.at[pl.ds(base, chunk)], dst_i)
            cp1 = pltpu.async_copy(xl_hbm.at[src_i], xl_b, sem1)
            cp2 = pltpu.async_copy(xr_hbm.at[dst_i], xr_b, sem2)
            cp1.wait()
            cp2.wait()

            if single_head:
                def edge_body(ei, _):
                    u = jnp.zeros((LANES,), jnp.float32)
                    xlv = []
                    for h in range(heads):
                        sl = pl.ds(h * LANES, LANES)
                        xlv.append(xl_b[ei, sl])
                        t = xlv[h] + xr_b[ei, sl]
                        t = jnp.maximum(t, NEG * t)
                        u = u + t * att_v[h, :]
                    p = jnp.exp(_vsum_bcast(u, iota))
                    for h in range(heads):
                        xl_b[ei, pl.ds(h * LANES, LANES)] = xlv[h] * p
                    den_b[ei, :] = jnp.where(iota == 0, p, 0.0)
                    return 0
            else:
                def edge_body(ei, _):
                    den_v = jnp.zeros((LANES,), jnp.float32)
                    for h in range(heads):
                        sl = pl.ds(h * LANES, LANES)
                        xlv = xl_b[ei, sl]
                        t = xlv + xr_b[ei, sl]
                        t = jnp.maximum(t, NEG * t)
                        p = jnp.exp(_vsum_bcast(t * att_v[h, :], iota))
                        xl_b[ei, sl] = xlv * p
                        den_v = jnp.where(iota == h, p, den_v)
                    den_b[ei, :] = den_v
                    return 0

            lax.fori_loop(0, chunk, edge_body, 0, unroll=False)
            pltpu.sync_copy(xl_b, num_sh.at[dst_i], add=True)
            pltpu.sync_copy(den_b, den_sh.at[dst_i], add=True)
            return 0

        lax.fori_loop(0, nchunks, chunk_body, 0, unroll=False)

        plsc.subcore_barrier()
        rw = (n // NS) & ~7
        rem = n - NS * rw
        off = s * rw
        pltpu.sync_copy(num_sh.at[pl.ds(off, rw)],
                        num_out.at[c, pl.ds(off, rw)])
        pltpu.sync_copy(den_sh.at[pl.ds(off, rw)],
                        den_out.at[c, pl.ds(off, rw)])

        @pl.when(s == NS - 1)
        def _():
            pltpu.sync_copy(num_sh.at[pl.ds(NS * rw, rem)],
                            num_out.at[c, pl.ds(NS * rw, rem)])
            pltpu.sync_copy(den_sh.at[pl.ds(NS * rw, rem)],
                            den_out.at[c, pl.ds(NS * rw, rem)])

    return k(xl, xr, src, dst, att)


def kernel(x, edge_index, Wl1, Wr1, att1, bias1, bn_w, bn_b, Wl2, Wr2, att2,
           bias2):
    n = x.shape[0]
    heads, hid = att1.shape

    A1 = (jnp.eye(heads, dtype=jnp.float32)[:, None, :]
          * att1[:, :, None]).reshape(heads * hid, heads)
    E8 = jnp.repeat(jnp.eye(heads, dtype=jnp.float32), hid, axis=1)
    bn_scale = bn_w / jnp.sqrt(1.0 + BN_EPS)
    scale = bn_scale.reshape(1, -1)
    shift = (bias1 * bn_scale + bn_b).reshape(1, -1)
    att2_sc = att2.reshape(-1, LANES)
    att2_tc = att2.reshape(1, -1)
    b2 = bias2.reshape(1, -1)

    src = edge_index[0]
    dst = edge_index[1]

    xl1, xr1 = _tc_lin(x, Wl1, Wr1)
    num1, den1 = _sc_edges(xl1, xr1, src, dst, att1, heads, 80, False)
    xl2, xr2 = _tc_combine1(
        xl1, xr1, num1[0], num1[1],
        den1[0, :, :heads], den1[1, :, :heads],
        A1, E8, scale, shift, Wl2, Wr2)

    num2, den2 = _sc_edges(xl2, xr2, src, dst, att2_sc, 4, 400, True)
    return _tc_final(xl2, xr2, num2[0], num2[1],
                     den2[0, :, 0:1], den2[1, :, 0:1],
                     att2_tc, b2)


# speedup vs baseline: 49.2742x; 2.6052x over previous
import functools

import jax
import jax.numpy as jnp
from jax import lax
from jax.experimental import pallas as pl
from jax.experimental.pallas import tpu as pltpu
from jax.experimental.pallas import tpu_sc as plsc

NEG = 0.2
BN_EPS = 1e-5
NC = 2
NS = 16
LANES = 16

_HI = lax.Precision.HIGHEST


def _tc_lin(x, Wl, Wr):
    n, din = x.shape
    dout = Wl.shape[1]
    bn = 1000
    grid = (n // bn,)

    def body(x_ref, wl_ref, wr_ref, xl_ref, xr_ref):
        xb = x_ref[...]
        xl_ref[...] = lax.dot(xb, wl_ref[...], precision=_HI,
                              preferred_element_type=jnp.float32)
        xr_ref[...] = lax.dot(xb, wr_ref[...], precision=_HI,
                              preferred_element_type=jnp.float32)

    return pl.pallas_call(
        body,
        grid=grid,
        in_specs=[
            pl.BlockSpec((bn, din), lambda i: (i, 0)),
            pl.BlockSpec((din, dout), lambda i: (0, 0)),
            pl.BlockSpec((din, dout), lambda i: (0, 0)),
        ],
        out_specs=[
            pl.BlockSpec((bn, dout), lambda i: (i, 0)),
            pl.BlockSpec((bn, dout), lambda i: (i, 0)),
        ],
        out_shape=[
            jax.ShapeDtypeStruct((n, dout), jnp.float32),
            jax.ShapeDtypeStruct((n, dout), jnp.float32),
        ],
    )(x, Wl, Wr)


def _tc_combine1(xl1, xr1, num0, num1, den0, den1, A1, E8, scale, shift,
                 Wl2, Wr2):
    n = xl1.shape[0]
    bn = 1000
    grid = (n // bn,)

    def body(xl_ref, xr_ref, n0_ref, n1_ref, d0_ref, d1_ref, a1_ref, e8_ref,
             sc_ref, sh_ref, wl2_ref, wr2_ref, xl2_ref, xr2_ref):
        xl = xl_ref[...]
        t = xl + xr_ref[...]
        t = jnp.maximum(t, NEG * t)
        p = jnp.exp(lax.dot(t, a1_ref[...], precision=_HI,
                            preferred_element_type=jnp.float32))
        den8 = d0_ref[...] + d1_ref[...] + p
        e8 = e8_ref[...]
        num = n0_ref[...] + n1_ref[...] + lax.dot(
            p, e8, precision=_HI, preferred_element_type=jnp.float32) * xl
        den = lax.dot(den8, e8, precision=_HI,
                      preferred_element_type=jnp.float32)
        o = num / (den + 1e-16)
        o = o * sc_ref[...] + sh_ref[...]
        h = jnp.where(o > 0, o, jnp.exp(o) - 1.0)
        xl2_ref[...] = lax.dot(h, wl2_ref[...], precision=_HI,
                               preferred_element_type=jnp.float32)
        xr2_ref[...] = lax.dot(h, wr2_ref[...], precision=_HI,
                               preferred_element_type=jnp.float32)

    full = lambda shape: pl.BlockSpec(shape, lambda i: tuple(0 for _ in shape))
    return pl.pallas_call(
        body,
        grid=grid,
        in_specs=[
            pl.BlockSpec((bn, 128), lambda i: (i, 0)),
            pl.BlockSpec((bn, 128), lambda i: (i, 0)),
            pl.BlockSpec((bn, 128), lambda i: (i, 0)),
            pl.BlockSpec((bn, 128), lambda i: (i, 0)),
            pl.BlockSpec((bn, 8), lambda i: (i, 0)),
            pl.BlockSpec((bn, 8), lambda i: (i, 0)),
            full((128, 8)),
            full((8, 128)),
            full((1, 128)),
            full((1, 128)),
            full((128, 64)),
            full((128, 64)),
        ],
        out_specs=[
            pl.BlockSpec((bn, 64), lambda i: (i, 0)),
            pl.BlockSpec((bn, 64), lambda i: (i, 0)),
        ],
        out_shape=[
            jax.ShapeDtypeStruct((n, 64), jnp.float32),
            jax.ShapeDtypeStruct((n, 64), jnp.float32),
        ],
    )(xl1, xr1, num0, num1, den0, den1, A1, E8, scale, shift, Wl2, Wr2)


def _tc_final(xl2, xr2, num0, num1, den0, den1, att2, bias2):
    n = xl2.shape[0]
    bn = 1000
    grid = (n // bn,)

    def body(xl_ref, xr_ref, n0_ref, n1_ref, d0_ref, d1_ref, a_ref, b_ref,
             out_ref):
        xl = xl_ref[...]
        t = xl + xr_ref[...]
        t = jnp.maximum(t, NEG * t)
        a = jnp.sum(t * a_ref[...], axis=1, keepdims=True)
        p = jnp.exp(a)
        den = d0_ref[...] + d1_ref[...] + p
        num = n0_ref[...] + n1_ref[...] + p * xl
        o = num / (den + 1e-16) + b_ref[...]
        m = jnp.max(o, axis=1, keepdims=True)
        lse = m + jnp.log(jnp.sum(jnp.exp(o - m), axis=1, keepdims=True))
        out_ref[...] = o - lse

    full = lambda shape: pl.BlockSpec(shape, lambda i: tuple(0 for _ in shape))
    return pl.pallas_call(
        body,
        grid=grid,
        in_specs=[
            pl.BlockSpec((bn, 64), lambda i: (i, 0)),
            pl.BlockSpec((bn, 64), lambda i: (i, 0)),
            pl.BlockSpec((bn, 64), lambda i: (i, 0)),
            pl.BlockSpec((bn, 64), lambda i: (i, 0)),
            pl.BlockSpec((bn, 1), lambda i: (i, 0)),
            pl.BlockSpec((bn, 1), lambda i: (i, 0)),
            full((1, 64)),
            full((1, 64)),
        ],
        out_specs=pl.BlockSpec((bn, 64), lambda i: (i, 0)),
        out_shape=jax.ShapeDtypeStruct((n, 64), jnp.float32),
    )(xl2, xr2, num0, num1, den0, den1, att2, bias2)


_GDN = lax.GatherDimensionNumbers(
    offset_dims=(), collapsed_slice_dims=(0,), start_index_map=(0,))


def _lane_shuffle(u, idx):
    return lax.gather(u, idx[:, None], _GDN, (1,),
                      mode=lax.GatherScatterMode.PROMISE_IN_BOUNDS)


def _vsum_bcast(u, iota):
    for k in (1, 2, 4, 8):
        u = u + _lane_shuffle(u, iota ^ k)
    return u

def _sc_edges(xl, xr, src, dst, att, heads, chunk, single_head):
    n = xl.shape[0]
    dim = heads * LANES
    e = src.shape[0]
    ew = e // (NC * NS)
    nchunks = ew // chunk
    rows = n // NS

    mesh = plsc.VectorSubcoreMesh(core_axis_name="c", subcore_axis_name="s")

    @functools.partial(
        pl.kernel,
        out_type=[
            jax.ShapeDtypeStruct((NC, n, dim), jnp.float32),
            jax.ShapeDtypeStruct((NC, n, LANES), jnp.float32),
        ],
        mesh=mesh,
        compiler_params=pltpu.CompilerParams(use_tc_tiling_on_sc=False),
        scratch_types=[
            pltpu.VMEM((chunk,), jnp.int32),
            pltpu.VMEM((chunk,), jnp.int32),
            pltpu.VMEM((chunk, dim), jnp.float32),
            pltpu.VMEM((chunk, dim), jnp.float32),
            pltpu.VMEM((chunk, LANES), jnp.float32),
            pltpu.VMEM((heads, LANES), jnp.float32),
            pltpu.VMEM_SHARED((n, dim), jnp.float32),
            pltpu.VMEM_SHARED((n, LANES), jnp.float32),
            pltpu.SemaphoreType.DMA,
            pltpu.SemaphoreType.DMA,
        ],
    )
    def k(xl_hbm, xr_hbm, src_hbm, dst_hbm, att_hbm, num_out, den_out,
          src_i, dst_i, xl_b, xr_b, den_b, att_v,
          num_sh, den_sh, sem1, sem2):
        c = lax.axis_index("c")
        s = lax.axis_index("s")
        wid = c * NS + s

        pltpu.sync_copy(att_hbm, att_v)

        zero = jnp.zeros((LANES,), jnp.float32)

        def zrow(i, _):
            for j in range(dim // LANES):
                xl_b[i, pl.ds(j * LANES, LANES)] = zero
            den_b[i, :] = zero
            return 0

        lax.fori_loop(0, chunk, zrow, 0, unroll=False)

        nf, zrem = divmod(rows, chunk)

        def zcopy(i, _):
            pltpu.sync_copy(xl_b, num_sh.at[pl.ds(s * rows + i * chunk,
                                                  chunk)])
            pltpu.sync_copy(den_b, den_sh.at[pl.ds(s * rows + i * chunk,
                                                   chunk)])
            return 0

        lax.fori_loop(0, nf, zcopy, 0, unroll=False)
        if zrem:
            pltpu.sync_copy(xl_b.at[pl.ds(0, zrem)],
                            num_sh.at[pl.ds(s * rows + nf * chunk, zrem)])
            pltpu.sync_copy(den_b.at[pl.ds(0, zrem)],
                            den_sh.at[pl.ds(s * rows + nf * chunk, zrem)])
        plsc.subcore_barrier()

        iota = lax.iota(jnp.int32, LANES)
        base_w = wid * ew

        def chunk_body(ci, _):
            base = base_w + ci * chunk
            pltpu.sync_copy(src_hbm.at[pl.ds(base, chunk)], src_i)
            pltpu.sync_copy(dst_hbm.at[pl.ds(base, chunk)], dst_i)
            cp1 = pltpu.async_copy(xl_hbm.at[src_i], xl_b, sem1)
            cp2 = pltpu.async_copy(xr_hbm.at[dst_i], xr_b, sem2)
            cp1.wait()
            cp2.wait()

            if single_head:
                @plsc.parallel_loop(0, chunk, step=1, unroll=4)
                def edge_body(ei):
                    u = jnp.zeros((LANES,), jnp.float32)
                    xlv = []
                    for h in range(heads):
                        sl = pl.ds(h * LANES, LANES)
                        xlv.append(xl_b[ei, sl])
                        t = xlv[h] + xr_b[ei, sl]
                        t = jnp.maximum(t, NEG * t)
                        u = u + t * att_v[h, :]
                    p = jnp.exp(_vsum_bcast(u, iota))
                    for h in range(heads):
                        xl_b[ei, pl.ds(h * LANES, LANES)] = xlv[h] * p
                    den_b[ei, :] = jnp.where(iota == 0, p, 0.0)
            else:
                @plsc.parallel_loop(0, chunk, step=1, unroll=4)
                def edge_body(ei):
                    den_v = jnp.zeros((LANES,), jnp.float32)
                    for h in range(heads):
                        sl = pl.ds(h * LANES, LANES)
                        xlv = xl_b[ei, sl]
                        t = xlv + xr_b[ei, sl]
                        t = jnp.maximum(t, NEG * t)
                        p = jnp.exp(_vsum_bcast(t * att_v[h, :], iota))
                        xl_b[ei, sl] = xlv * p
                        den_v = jnp.where(iota == h, p, den_v)
                    den_b[ei, :] = den_v
            pltpu.sync_copy(xl_b, num_sh.at[dst_i], add=True)
            pltpu.sync_copy(den_b, den_sh.at[dst_i], add=True)
            return 0

        lax.fori_loop(0, nchunks, chunk_body, 0, unroll=False)

        plsc.subcore_barrier()
        rw = (n // NS) & ~7
        rem = n - NS * rw
        off = s * rw
        pltpu.sync_copy(num_sh.at[pl.ds(off, rw)],
                        num_out.at[c, pl.ds(off, rw)])
        pltpu.sync_copy(den_sh.at[pl.ds(off, rw)],
                        den_out.at[c, pl.ds(off, rw)])

        @pl.when(s == NS - 1)
        def _():
            pltpu.sync_copy(num_sh.at[pl.ds(NS * rw, rem)],
                            num_out.at[c, pl.ds(NS * rw, rem)])
            pltpu.sync_copy(den_sh.at[pl.ds(NS * rw, rem)],
                            den_out.at[c, pl.ds(NS * rw, rem)])

    return k(xl, xr, src, dst, att)


def kernel(x, edge_index, Wl1, Wr1, att1, bias1, bn_w, bn_b, Wl2, Wr2, att2,
           bias2):
    n = x.shape[0]
    heads, hid = att1.shape

    A1 = (jnp.eye(heads, dtype=jnp.float32)[:, None, :]
          * att1[:, :, None]).reshape(heads * hid, heads)
    E8 = jnp.repeat(jnp.eye(heads, dtype=jnp.float32), hid, axis=1)
    bn_scale = bn_w / jnp.sqrt(1.0 + BN_EPS)
    scale = bn_scale.reshape(1, -1)
    shift = (bias1 * bn_scale + bn_b).reshape(1, -1)
    att2_sc = att2.reshape(-1, LANES)
    att2_tc = att2.reshape(1, -1)
    b2 = bias2.reshape(1, -1)

    src = edge_index[0]
    dst = edge_index[1]

    xl1, xr1 = _tc_lin(x, Wl1, Wr1)
    num1, den1 = _sc_edges(xl1, xr1, src, dst, att1, heads, 80, False)
    xl2, xr2 = _tc_combine1(
        xl1, xr1, num1[0], num1[1],
        den1[0, :, :heads], den1[1, :, :heads],
        A1, E8, scale, shift, Wl2, Wr2)

    num2, den2 = _sc_edges(xl2, xr2, src, dst, att2_sc, 4, 400, True)
    return _tc_final(xl2, xr2, num2[0], num2[1],
                     den2[0, :, 0:1], den2[1, :, 0:1],
                     att2_tc, b2)
